# Initial kernel scaffold; baseline (speedup 1.0000x reference)
import jax
import jax.numpy as jnp
from jax.experimental import pallas as pl


def kernel(x, weights, w3, w1):
    raise NotImplementedError("write your pallas kernel here")


# speedup vs baseline: 3.3860x; 3.3860x over previous
import functools

import jax
import jax.numpy as jnp
from jax.experimental import pallas as pl
from jax.experimental.pallas import tpu as pltpu

_NEG = -1e30


def _mixed_down_body(s_ref, xph_ref, wm_ref, aux_ref, o_ref, *, C, P, Wo):
    w_max = s_ref[1]
    avg_scale = aux_ref[0:1, :]
    rmask = aux_ref[1:2, :]
    cmask = aux_ref[2:3, :]
    rcmask = aux_ref[3:4, :]
    rpen = aux_ref[4:5, :]
    cpen = aux_ref[5:6, :]
    rcpen = aux_ref[6:7, :]

    x00 = xph_ref[0, 0]
    x01 = xph_ref[0, 1]
    x10 = xph_ref[0, 2]
    x11 = xph_ref[0, 3]

    def shr(a, k):
        return jnp.concatenate([jnp.zeros((C, k), a.dtype), a[:, :-k]], axis=1)

    t10 = shr(x01, 1) * cmask
    t01 = shr(x10, Wo) * rmask
    t20 = shr(x11, 1) * cmask
    t02 = shr(x11, Wo) * rmask
    t00 = shr(x11, Wo + 1) * rcmask

    taps = (x00, x01, x10, x11, t10, t01, t20, t02, t00)
    pens = (None, None, None, None, cpen, rpen, cpen, rpen, rcpen)

    acc = None
    ssum = None
    mxv = None
    for i in range(9):
        t = taps[i]
        d = jnp.dot(wm_ref[i], t.astype(jnp.bfloat16),
                    preferred_element_type=jnp.float32)
        acc = d if acc is None else acc + d
        ssum = t if ssum is None else ssum + t
        m = t if pens[i] is None else t + pens[i]
        mxv = m if mxv is None else jnp.maximum(mxv, m)

    o_ref[0] = acc + w_max * mxv + ssum * avg_scale


def kernel(x, weights, w3, w1):
    N, C, H, W = x.shape
    assert H % 2 == 0 and W % 2 == 0, "stride-2 downsample expects even H, W"
    Ho, Wo = H // 2, W // 2
    P = Ho * Wo

    xf = x.astype(jnp.float32)
    w_ops = weights.astype(jnp.float32)
    w3f = w3.astype(jnp.float32)
    w1f = w1.astype(jnp.float32).reshape(C, C)

    xph = xf.reshape(N, C, Ho, 2, Wo, 2)
    xph = jnp.transpose(xph, (0, 3, 5, 1, 2, 4)).reshape(N, 4, C, P)

    wt = jnp.transpose(w3f, (2, 3, 0, 1)) * w_ops[2]
    wm = jnp.stack([
        wt[1, 1] + w_ops[3] * w1f,
        wt[1, 2], wt[2, 1], wt[2, 2],
        wt[1, 0], wt[0, 1], wt[2, 0], wt[0, 2], wt[0, 0],
    ]).astype(jnp.bfloat16)

    row0 = (jnp.arange(Ho) == 0).astype(jnp.float32)
    col0 = (jnp.arange(Wo) == 0).astype(jnp.float32)
    count = (3.0 - row0)[:, None] * (3.0 - col0)[None, :]
    avg_scale = (w_ops[0] / count).reshape(P)
    rmask = jnp.broadcast_to((1.0 - row0)[:, None], (Ho, Wo)).reshape(P)
    cmask = jnp.broadcast_to((1.0 - col0)[None, :], (Ho, Wo)).reshape(P)
    rcmask = rmask * cmask
    aux = jnp.stack([
        avg_scale, rmask, cmask, rcmask,
        _NEG * (1.0 - rmask), _NEG * (1.0 - cmask), _NEG * (1.0 - rcmask),
        jnp.zeros((P,), jnp.float32),
    ])

    body = functools.partial(_mixed_down_body, C=C, P=P, Wo=Wo)
    out = pl.pallas_call(
        body,
        out_shape=jax.ShapeDtypeStruct((N, C, P), jnp.float32),
        grid_spec=pltpu.PrefetchScalarGridSpec(
            num_scalar_prefetch=1,
            grid=(N,),
            in_specs=[
                pl.BlockSpec((1, 4, C, P), lambda n, s: (n, 0, 0, 0)),
                pl.BlockSpec((9, C, C), lambda n, s: (0, 0, 0)),
                pl.BlockSpec((8, P), lambda n, s: (0, 0)),
            ],
            out_specs=pl.BlockSpec((1, C, P), lambda n, s: (n, 0, 0)),
        ),
        compiler_params=pltpu.CompilerParams(
            dimension_semantics=("parallel",),
            vmem_limit_bytes=64 * 1024 * 1024,
        ),
    )(w_ops, xph, wm, aux)

    return out.reshape(N, C, Ho, Wo)


# speedup vs baseline: 3.3882x; 1.0006x over previous
import functools

import jax
import jax.numpy as jnp
from jax.experimental import pallas as pl
from jax.experimental.pallas import tpu as pltpu

_NEG = -1e30


def _mixed_down_body(s_ref, x_ref, sel_ref, wm_ref, aux_ref, o_ref, *, C, H, W):
    Ho, Wo = H // 2, W // 2
    P = Ho * Wo
    w_max = s_ref[1]
    avg_scale = aux_ref[0:1, :]
    rmask = aux_ref[1:2, :]
    cmask = aux_ref[2:3, :]
    rcmask = aux_ref[3:4, :]
    rpen = aux_ref[4:5, :]
    cpen = aux_ref[5:6, :]
    rcpen = aux_ref[6:7, :]

    sel = sel_ref[:, :]

    def phases(rows):
        y = jnp.dot(rows.reshape(C * Ho, W).astype(jnp.bfloat16), sel,
                    preferred_element_type=jnp.float32)
        y = y.reshape(C, Ho, W)
        return (y[:, :, 0:Wo].reshape(C, P), y[:, :, Wo:W].reshape(C, P))

    x00, x01 = phases(x_ref[0, :, 0::2, :])
    x10, x11 = phases(x_ref[0, :, 1::2, :])

    def shr(a, k):
        return jnp.concatenate([jnp.zeros((C, k), a.dtype), a[:, :-k]], axis=1)

    t10 = shr(x01, 1) * cmask
    t01 = shr(x10, Wo) * rmask
    t20 = shr(x11, 1) * cmask
    t02 = shr(x11, Wo) * rmask
    t00 = shr(x11, Wo + 1) * rcmask

    taps = (x00, x01, x10, x11, t10, t01, t20, t02, t00)
    pens = (None, None, None, None, cpen, rpen, cpen, rpen, rcpen)

    acc = None
    ssum = None
    mxv = None
    for i in range(9):
        t = taps[i]
        d = jnp.dot(wm_ref[i], t.astype(jnp.bfloat16),
                    preferred_element_type=jnp.float32)
        acc = d if acc is None else acc + d
        ssum = t if ssum is None else ssum + t
        m = t if pens[i] is None else t + pens[i]
        mxv = m if mxv is None else jnp.maximum(mxv, m)

    o_ref[0] = acc + w_max * mxv + ssum * avg_scale


def kernel(x, weights, w3, w1):
    N, C, H, W = x.shape
    assert H % 2 == 0 and W % 2 == 0, "stride-2 downsample expects even H, W"
    Ho, Wo = H // 2, W // 2
    P = Ho * Wo

    xf = x.astype(jnp.float32)
    w_ops = weights.astype(jnp.float32)
    w3f = w3.astype(jnp.float32)
    w1f = w1.astype(jnp.float32).reshape(C, C)

    w_idx = jnp.arange(W)
    j_idx = jnp.arange(W)
    sel = ((j_idx[None, :] < Wo) & (w_idx[:, None] == 2 * j_idx[None, :])) | \
          ((j_idx[None, :] >= Wo) & (w_idx[:, None] == 2 * (j_idx[None, :] - Wo) + 1))
    sel = sel.astype(jnp.bfloat16)

    wt = jnp.transpose(w3f, (2, 3, 0, 1)) * w_ops[2]
    wm = jnp.stack([
        wt[1, 1] + w_ops[3] * w1f,
        wt[1, 2], wt[2, 1], wt[2, 2],
        wt[1, 0], wt[0, 1], wt[2, 0], wt[0, 2], wt[0, 0],
    ]).astype(jnp.bfloat16)

    row0 = (jnp.arange(Ho) == 0).astype(jnp.float32)
    col0 = (jnp.arange(Wo) == 0).astype(jnp.float32)
    count = (3.0 - row0)[:, None] * (3.0 - col0)[None, :]
    avg_scale = (w_ops[0] / count).reshape(P)
    rmask = jnp.broadcast_to((1.0 - row0)[:, None], (Ho, Wo)).reshape(P)
    cmask = jnp.broadcast_to((1.0 - col0)[None, :], (Ho, Wo)).reshape(P)
    rcmask = rmask * cmask
    aux = jnp.stack([
        avg_scale, rmask, cmask, rcmask,
        _NEG * (1.0 - rmask), _NEG * (1.0 - cmask), _NEG * (1.0 - rcmask),
        jnp.zeros((P,), jnp.float32),
    ])

    body = functools.partial(_mixed_down_body, C=C, H=H, W=W)
    out = pl.pallas_call(
        body,
        out_shape=jax.ShapeDtypeStruct((N, C, P), jnp.float32),
        grid_spec=pltpu.PrefetchScalarGridSpec(
            num_scalar_prefetch=1,
            grid=(N,),
            in_specs=[
                pl.BlockSpec((1, C, H, W), lambda n, s: (n, 0, 0, 0)),
                pl.BlockSpec((W, W), lambda n, s: (0, 0)),
                pl.BlockSpec((9, C, C), lambda n, s: (0, 0, 0)),
                pl.BlockSpec((8, P), lambda n, s: (0, 0)),
            ],
            out_specs=pl.BlockSpec((1, C, P), lambda n, s: (n, 0, 0)),
        ),
        compiler_params=pltpu.CompilerParams(
            dimension_semantics=("parallel",),
            vmem_limit_bytes=64 * 1024 * 1024,
        ),
    )(w_ops, xf, sel, wm, aux)

    return out.reshape(N, C, Ho, Wo)


# speedup vs baseline: 3.8728x; 1.1430x over previous
import functools

import jax
import jax.numpy as jnp
from jax.experimental import pallas as pl
from jax.experimental.pallas import tpu as pltpu

_NEG = -1e30


def _mixed_down_body(s_ref, x_ref, sel_ref, wm_ref, aux_ref, o_ref, *, C, H, W):
    Ho, Wo = H // 2, W // 2
    P = Ho * Wo
    w_max = s_ref[1]
    avg_scale = aux_ref[0:1, :]
    rmask = aux_ref[1:2, :]
    cmask = aux_ref[2:3, :]
    rcmask = aux_ref[3:4, :]
    rpen = aux_ref[4:5, :]
    cpen = aux_ref[5:6, :]
    rcpen = aux_ref[6:7, :]

    sel = sel_ref[:, :]

    def phases(rows):
        y = jnp.dot(rows.reshape(C * Ho, W).astype(jnp.bfloat16), sel,
                    preferred_element_type=jnp.float32)
        y = y.reshape(C, Ho, W)
        return (y[:, :, 0:Wo].reshape(C, P), y[:, :, Wo:W].reshape(C, P))

    v = x_ref[0].reshape(C, Ho, 2 * W)
    x00, x01 = phases(v[:, :, 0:W])
    x10, x11 = phases(v[:, :, W:2 * W])

    def shr(a, k):
        return jnp.concatenate([jnp.zeros((C, k), a.dtype), a[:, :-k]], axis=1)

    t10 = shr(x01, 1) * cmask
    t01 = shr(x10, Wo) * rmask
    t20 = shr(x11, 1) * cmask
    t02 = shr(x11, Wo) * rmask
    t00 = shr(x11, Wo + 1) * rcmask

    taps = (x00, x01, x10, x11, t10, t01, t20, t02, t00)
    pens = (None, None, None, None, cpen, rpen, cpen, rpen, rcpen)

    acc = None
    ssum = None
    mxv = None
    for i in range(9):
        t = taps[i]
        d = jnp.dot(wm_ref[i], t.astype(jnp.bfloat16),
                    preferred_element_type=jnp.float32)
        acc = d if acc is None else acc + d
        ssum = t if ssum is None else ssum + t
        m = t if pens[i] is None else t + pens[i]
        mxv = m if mxv is None else jnp.maximum(mxv, m)

    o_ref[0] = acc + w_max * mxv + ssum * avg_scale


def kernel(x, weights, w3, w1):
    N, C, H, W = x.shape
    assert H % 2 == 0 and W % 2 == 0, "stride-2 downsample expects even H, W"
    Ho, Wo = H // 2, W // 2
    P = Ho * Wo

    xf = x.astype(jnp.float32)
    w_ops = weights.astype(jnp.float32)
    w3f = w3.astype(jnp.float32)
    w1f = w1.astype(jnp.float32).reshape(C, C)

    w_idx = jnp.arange(W)
    j_idx = jnp.arange(W)
    sel = ((j_idx[None, :] < Wo) & (w_idx[:, None] == 2 * j_idx[None, :])) | \
          ((j_idx[None, :] >= Wo) & (w_idx[:, None] == 2 * (j_idx[None, :] - Wo) + 1))
    sel = sel.astype(jnp.bfloat16)

    wt = jnp.transpose(w3f, (2, 3, 0, 1)) * w_ops[2]
    wm = jnp.stack([
        wt[1, 1] + w_ops[3] * w1f,
        wt[1, 2], wt[2, 1], wt[2, 2],
        wt[1, 0], wt[0, 1], wt[2, 0], wt[0, 2], wt[0, 0],
    ]).astype(jnp.bfloat16)

    row0 = (jnp.arange(Ho) == 0).astype(jnp.float32)
    col0 = (jnp.arange(Wo) == 0).astype(jnp.float32)
    count = (3.0 - row0)[:, None] * (3.0 - col0)[None, :]
    avg_scale = (w_ops[0] / count).reshape(P)
    rmask = jnp.broadcast_to((1.0 - row0)[:, None], (Ho, Wo)).reshape(P)
    cmask = jnp.broadcast_to((1.0 - col0)[None, :], (Ho, Wo)).reshape(P)
    rcmask = rmask * cmask
    aux = jnp.stack([
        avg_scale, rmask, cmask, rcmask,
        _NEG * (1.0 - rmask), _NEG * (1.0 - cmask), _NEG * (1.0 - rcmask),
        jnp.zeros((P,), jnp.float32),
    ])

    body = functools.partial(_mixed_down_body, C=C, H=H, W=W)
    out = pl.pallas_call(
        body,
        out_shape=jax.ShapeDtypeStruct((N, C, P), jnp.float32),
        grid_spec=pltpu.PrefetchScalarGridSpec(
            num_scalar_prefetch=1,
            grid=(N,),
            in_specs=[
                pl.BlockSpec((1, C, H * W), lambda n, s: (n, 0, 0)),
                pl.BlockSpec((W, W), lambda n, s: (0, 0)),
                pl.BlockSpec((9, C, C), lambda n, s: (0, 0, 0)),
                pl.BlockSpec((8, P), lambda n, s: (0, 0)),
            ],
            out_specs=pl.BlockSpec((1, C, P), lambda n, s: (n, 0, 0)),
        ),
        compiler_params=pltpu.CompilerParams(
            dimension_semantics=("parallel",),
            vmem_limit_bytes=64 * 1024 * 1024,
        ),
    )(w_ops, xf.reshape(N, C, H * W), sel, wm, aux)

    return out.reshape(N, C, Ho, Wo)


# speedup vs baseline: 17.4402x; 4.5033x over previous
import functools

import jax
import jax.numpy as jnp
from jax.experimental import pallas as pl
from jax.experimental.pallas import tpu as pltpu

_NEG = -1e30


def _mixed_down_body(s_ref, x_ref, wm_ref, o_ref, *, BN, H, W, C):
    Ho, Wo = H // 2, W // 2
    P = Ho * Wo
    M = BN * P
    w_avg = s_ref[0]
    w_max = s_ref[1]

    pidx = jax.lax.broadcasted_iota(jnp.int32, (M, 1), 0)
    ho0 = pidx % P < Wo
    wo0 = pidx % Wo == 0
    rmask = jnp.where(ho0, 0.0, 1.0)
    cmask = jnp.where(wo0, 0.0, 1.0)
    rcmask = rmask * cmask
    rpen = jnp.where(ho0, _NEG, 0.0)
    cpen = jnp.where(wo0, _NEG, 0.0)
    rcpen = rpen + cpen
    avg_scale = w_avg / ((2.0 + rmask) * (2.0 + cmask))

    x00 = x_ref[:, 0::2, 0::2, :].reshape(M, C)
    x01 = x_ref[:, 0::2, 1::2, :].reshape(M, C)
    x10 = x_ref[:, 1::2, 0::2, :].reshape(M, C)
    x11 = x_ref[:, 1::2, 1::2, :].reshape(M, C)

    def shr(a, k):
        return jnp.concatenate([jnp.zeros((k, C), a.dtype), a[:-k]], axis=0)

    t10 = shr(x01, 1) * cmask
    t01 = shr(x10, Wo) * rmask
    t20 = shr(x11, 1) * cmask
    t02 = shr(x11, Wo) * rmask
    t00 = shr(x11, Wo + 1) * rcmask

    taps = (t00, t01, t02, t10, x00, x01, t20, x10, x11)
    pens = (rcpen, rpen, rpen, cpen, None, None, cpen, None, None)

    acc = None
    ssum = None
    mxv = None
    for i in range(9):
        t = taps[i]
        d = jax.lax.dot_general(t.astype(jnp.bfloat16), wm_ref[i],
                                (((1,), (1,)), ((), ())),
                                preferred_element_type=jnp.float32)
        acc = d if acc is None else acc + d
        ssum = t if ssum is None else ssum + t
        m = t if pens[i] is None else t + pens[i]
        mxv = m if mxv is None else jnp.maximum(mxv, m)

    res = acc + w_max * mxv + ssum * avg_scale
    o_ref[...] = res.reshape(BN, P, C)


def kernel(x, weights, w3, w1):
    N, C, H, W = x.shape
    assert H % 2 == 0 and W % 2 == 0, "stride-2 downsample expects even H, W"
    Ho, Wo = H // 2, W // 2
    P = Ho * Wo

    xt = jnp.transpose(x.astype(jnp.float32), (0, 2, 3, 1))
    w_ops = weights.astype(jnp.float32)
    w3f = w3.astype(jnp.float32)
    w1f = w1.astype(jnp.float32).reshape(C, C)

    wt = jnp.transpose(w3f.reshape(C, C, 9), (2, 0, 1)) * w_ops[2]
    center = (jnp.arange(9) == 4).astype(jnp.float32).reshape(9, 1, 1)
    wm = (wt + center * (w_ops[3] * w1f)).astype(jnp.bfloat16)

    BN = 2 if N % 2 == 0 else 1
    body = functools.partial(_mixed_down_body, BN=BN, H=H, W=W, C=C)
    out = pl.pallas_call(
        body,
        out_shape=jax.ShapeDtypeStruct((N, P, C), jnp.float32),
        grid_spec=pltpu.PrefetchScalarGridSpec(
            num_scalar_prefetch=1,
            grid=(N // BN,),
            in_specs=[
                pl.BlockSpec((BN, H, W, C), lambda n, s: (n, 0, 0, 0)),
                pl.BlockSpec((9, C, C), lambda n, s: (0, 0, 0)),
            ],
            out_specs=pl.BlockSpec((BN, P, C), lambda n, s: (n, 0, 0)),
        ),
        compiler_params=pltpu.CompilerParams(
            dimension_semantics=("parallel",),
            vmem_limit_bytes=64 * 1024 * 1024,
        ),
    )(w_ops, xt, wm)

    return jnp.transpose(out.reshape(N, Ho, Wo, C), (0, 3, 1, 2))
